# baseline (speedup 1.0000x reference)
import functools

import jax
import jax.numpy as jnp
from jax import lax
from jax.experimental import pallas as pl
from jax.experimental.pallas import tpu as pltpu
from jax.experimental.pallas import tpu_sc as plsc

B = 16384
NW = 32
BPW = B // NW
CHUNK = 64
NCHUNK = BPW // CHUNK
L_UID = 8192
L_SMALL = 8192


def _pack(x):
    d, l = x.shape
    p = 128 // d
    l1 = l // p
    eye = jnp.eye(d, dtype=jnp.float32)
    xt = lax.dot_general(x, eye, (((0,), (0,)), ((), ())),
                         preferred_element_type=jnp.float32)
    return jnp.concatenate([xt[j * l1:(j + 1) * l1] for j in range(p)], axis=1)


def _tp_body_bfpack(in_ref, out_ref):
    d, l = in_ref.shape
    eye = jnp.eye(d, dtype=jnp.float32)
    xt = lax.dot_general(in_ref[...], eye, (((0,), (0,)), ((), ())),
                         preferred_element_type=jnp.float32)
    bits = lax.bitcast_convert_type(xt, jnp.int32)
    lo = lax.shift_right_logical(bits[:, 0:32] + jnp.int32(0x8000), 16)
    hi = lax.bitwise_and(bits[:, 32:64] + jnp.int32(0x8000),
                         jnp.int32(-65536))
    w = lax.bitwise_or(lo, hi)
    l1 = l // 4
    out_ref[...] = jnp.concatenate(
        [w[j * l1:(j + 1) * l1] for j in range(4)], axis=1)


def _transpose_pack_uid(table_t, lanes):
    d, v = table_t.shape
    grid = (v + lanes - 1) // lanes
    rows = lanes // 4
    return pl.pallas_call(
        _tp_body_bfpack,
        grid=(grid,),
        in_specs=[pl.BlockSpec((d, lanes), lambda i: (0, i))],
        out_specs=pl.BlockSpec((rows, 128), lambda i: (i, 0)),
        out_shape=jax.ShapeDtypeStruct((grid * rows, 128), jnp.int32),
    )(table_t)


def _tp_body6(a_ref, b_ref, c_ref, d_ref, e_ref, f_ref, ao, bo, co, do_, eo,
              fo):
    refs = (a_ref, b_ref, c_ref, d_ref, e_ref, f_ref)
    x_all = jnp.concatenate([r[...] for r in refs], axis=0)
    k, l = x_all.shape
    eye = jnp.eye(k, dtype=jnp.float32)
    xt = lax.dot_general(x_all, eye, (((0,), (0,)), ((), ())),
                         preferred_element_type=jnp.float32)
    col = 0
    for r, o in zip(refs, (ao, bo, co, do_, eo, fo)):
        d = r.shape[0]
        p = 128 // d
        l1 = l // p
        o[...] = jnp.concatenate(
            [xt[j * l1:(j + 1) * l1, col:col + d] for j in range(p)], axis=1)
        col += d


def _transpose_pack6(tables_t, lanes):
    v = tables_t[0].shape[1]
    grid = (v + lanes - 1) // lanes
    outs, in_specs, out_specs = [], [], []
    for t in tables_t:
        d = t.shape[0]
        rows = lanes * d // 128
        outs.append(jax.ShapeDtypeStruct((grid * rows, 128), jnp.float32))
        in_specs.append(pl.BlockSpec((d, lanes), lambda i: (0, i)))
        out_specs.append(pl.BlockSpec((rows, 128), lambda i: (i, 0)))
    return pl.pallas_call(
        _tp_body6,
        grid=(grid,),
        in_specs=in_specs,
        out_specs=out_specs,
        out_shape=outs,
    )(*tables_t)


_MESH = plsc.VectorSubcoreMesh(core_axis_name="c", subcore_axis_name="s")


def _extract(gbuf, off, r, width, asm, col0):
    for k in range(width // 16):
        asm[r, pl.ds(col0 + 16 * k, 16)] = gbuf[r, pl.ds(off + 16 * k, 16)]


def _sc_small_body(idx_hbm, off_hbm, seg_t, beh_t, act_t, typ_t, des_t,
                   u_out, i_out, idxv, offv, g0, g1, g2, g3, g4, au, ai, sem):
    wid = lax.axis_index("s") * 2 + lax.axis_index("c")
    base = wid * BPW

    z16 = jnp.zeros((16,), jnp.float32)
    for r in range(CHUNK):
        au[r, pl.ds(48, 16)] = z16
        ai[r, pl.ds(32, 16)] = z16
        ai[r, pl.ds(48, 16)] = z16

    def chunk_body(c, carry):
        pltpu.sync_copy(idx_hbm.at[wid, c], idxv)
        pltpu.sync_copy(off_hbm.at[wid, c], offv)
        copies = [
            pltpu.async_copy(seg_t.at[idxv.at[0]], g0, sem),
            pltpu.async_copy(beh_t.at[idxv.at[1]], g1, sem),
            pltpu.async_copy(act_t.at[idxv.at[2]], g2, sem),
            pltpu.async_copy(typ_t.at[idxv.at[3]], g3, sem),
            pltpu.async_copy(des_t.at[idxv.at[4]], g4, sem),
        ]
        for cp in copies:
            cp.wait()
        for g in range(CHUNK // 16):
            ovs = [offv[f, pl.ds(g * 16, 16)] for f in range(5)]
            for rr in range(16):
                r = g * 16 + rr
                _extract(g0, ovs[0][rr], r, 16, au, 0)
                _extract(g1, ovs[1][rr], r, 16, au, 16)
                _extract(g2, ovs[2][rr], r, 16, au, 32)
                _extract(g3, ovs[3][rr], r, 16, ai, 0)
                _extract(g4, ovs[4][rr], r, 16, ai, 16)
        row0 = pl.multiple_of(base + c * CHUNK, CHUNK)
        pltpu.sync_copy(au, u_out.at[pl.ds(row0, CHUNK)])
        pltpu.sync_copy(ai, i_out.at[pl.ds(row0, CHUNK)])
        return carry

    lax.fori_loop(0, NCHUNK, chunk_body, 0)


_sc_small = functools.partial(
    pl.kernel,
    mesh=_MESH,
    out_type=[
        jax.ShapeDtypeStruct((B, 64), jnp.float32),
        jax.ShapeDtypeStruct((B, 64), jnp.float32),
    ],
    scratch_types=[
        pltpu.VMEM((5, CHUNK), jnp.int32),
        pltpu.VMEM((5, CHUNK), jnp.int32),
        pltpu.VMEM((CHUNK, 128), jnp.float32),
        pltpu.VMEM((CHUNK, 128), jnp.float32),
        pltpu.VMEM((CHUNK, 128), jnp.float32),
        pltpu.VMEM((CHUNK, 128), jnp.float32),
        pltpu.VMEM((CHUNK, 128), jnp.float32),
        pltpu.VMEM((CHUNK, 64), jnp.float32),
        pltpu.VMEM((CHUNK, 64), jnp.float32),
        pltpu.SemaphoreType.DMA,
    ],
)(_sc_small_body)


def _sc_ids_body(idx_hbm, off_hbm, uid_t, iid_t, u_out, i_out,
                 idxv, offv, g0, g1, au, ai, sem):
    wid = lax.axis_index("s") * 2 + lax.axis_index("c")
    base = wid * BPW

    def chunk_body(c, carry):
        pltpu.sync_copy(idx_hbm.at[wid, c], idxv)
        pltpu.sync_copy(off_hbm.at[wid, c], offv)
        cu = pltpu.async_copy(uid_t.at[idxv.at[0]], g0, sem)
        ci = pltpu.async_copy(iid_t.at[idxv.at[1]], g1, sem)
        cu.wait()
        ci.wait()
        for g in range(CHUNK // 16):
            ov0 = offv[0, pl.ds(g * 16, 16)]
            ov1 = offv[1, pl.ds(g * 16, 16)]
            for rr in range(16):
                r = g * 16 + rr
                _extract(g0, ov0[rr], r, 32, au, 0)
                _extract(g1, ov1[rr], r, 64, ai, 0)
        row0 = pl.multiple_of(base + c * CHUNK, CHUNK)
        pltpu.sync_copy(au, u_out.at[pl.ds(row0, CHUNK)])
        pltpu.sync_copy(ai, i_out.at[pl.ds(row0, CHUNK)])
        return carry

    lax.fori_loop(0, NCHUNK, chunk_body, 0)


_sc_ids = functools.partial(
    pl.kernel,
    mesh=_MESH,
    out_type=[
        jax.ShapeDtypeStruct((B, 32), jnp.int32),
        jax.ShapeDtypeStruct((B, 64), jnp.float32),
    ],
    scratch_types=[
        pltpu.VMEM((2, CHUNK), jnp.int32),
        pltpu.VMEM((2, CHUNK), jnp.int32),
        pltpu.VMEM((CHUNK, 128), jnp.int32),
        pltpu.VMEM((CHUNK, 128), jnp.float32),
        pltpu.VMEM((CHUNK, 32), jnp.int32),
        pltpu.VMEM((CHUNK, 64), jnp.float32),
        pltpu.SemaphoreType.DMA,
    ],
)(_sc_ids_body)


BM = 2048


def _tc_mlp_body(uid_r, us_r, iid_r, is_r, uWa_r, uWb_r, ub1_r, uW2_r, ub2_r,
                 iWa_r, iWb_r, ib1_r, iW2_r, ib2_r, o_ref):
    dot = functools.partial(jnp.dot, preferred_element_type=jnp.float32)
    w = uid_r[...]
    lo = lax.bitcast_convert_type(lax.shift_left(w, 16), jnp.float32)
    hi = lax.bitcast_convert_type(
        lax.bitwise_and(w, jnp.int32(-65536)), jnp.float32)
    uid64 = jnp.concatenate([lo, hi], axis=1)
    uh = jnp.maximum(
        dot(uid64, uWa_r[...])
        + dot(us_r[...], uWb_r[...])
        + ub1_r[...], 0.0)
    ur = dot(uh, uW2_r[...]) + ub2_r[...]
    ih = jnp.maximum(
        dot(iid_r[...], iWa_r[...]) + dot(is_r[...], iWb_r[...])
        + ib1_r[...], 0.0)
    ir = dot(ih, iW2_r[...]) + ib2_r[...]
    o_ref[...] = jnp.sum(ur * ir, axis=1)


def kernel(user_features_batch, item_features_batch, user_id_emb, seg_emb,
           beh_emb, act_emb, uW1, ub1, uW2, ub2, item_id_emb, type_emb,
           descrip_emb, iW1, ib1, iW2, ib2):
    iid_t, seg_t, beh_t, act_t, typ_t, des_t = _transpose_pack6(
        [item_id_emb.T, seg_emb.T, beh_emb.T, act_emb.T, type_emb.T,
         descrip_emb.T], L_SMALL)
    uid_t = _transpose_pack_uid(user_id_emb.T, L_UID)

    def _addr(idx, d, lanes):
        l1 = lanes * d // 128
        b, r = idx // lanes, idx % lanes
        return b * l1 + r % l1, (r // l1) * d

    u0, u1, u2, u3 = (user_features_batch[:, k] for k in range(4))
    i0, i1, i2 = (item_features_batch[:, k] for k in range(3))

    small = [_addr(u1, 16, L_SMALL), _addr(u2, 16, L_SMALL),
             _addr(u3, 16, L_SMALL), _addr(i1, 16, L_SMALL),
             _addr(i2, 16, L_SMALL)]
    ids = [_addr(u0, 32, L_UID), _addr(i0, 64, L_SMALL)]

    def _plan(addrs, n):
        rows = jnp.stack([a[0] for a in addrs])
        offs = jnp.stack([a[1] for a in addrs])
        return (rows.reshape(n, NW, NCHUNK, CHUNK).transpose(1, 2, 0, 3),
                offs.reshape(n, NW, NCHUNK, CHUNK).transpose(1, 2, 0, 3))

    idxS, offS = _plan(small, 5)
    idxI, offI = _plan(ids, 2)

    u_small, i_small = _sc_small(idxS, offS, seg_t, beh_t, act_t, typ_t, des_t)
    u_id, i_id = _sc_ids(idxI, offI, uid_t, iid_t)

    uWa = uW1[0:64]
    uWb = jnp.zeros((64, uW1.shape[1]), jnp.float32).at[0:48].set(uW1[64:112])
    iWa = iW1[0:64]
    iWb = jnp.zeros((64, iW1.shape[1]), jnp.float32).at[0:32].set(iW1[64:96])

    full = lambda a: pl.BlockSpec(a.shape, lambda i: tuple(0 for _ in a.shape))
    feat = pl.BlockSpec((BM, 64), lambda i: (i, 0))
    featp = pl.BlockSpec((BM, 32), lambda i: (i, 0))
    out = pl.pallas_call(
        _tc_mlp_body,
        grid=(B // BM,),
        in_specs=[featp, feat, feat, feat,
                  full(uWa), full(uWb), pl.BlockSpec((1, 56), lambda i: (0, 0)),
                  full(uW2), pl.BlockSpec((1, 64), lambda i: (0, 0)),
                  full(iWa), full(iWb), pl.BlockSpec((1, 48), lambda i: (0, 0)),
                  full(iW2), pl.BlockSpec((1, 64), lambda i: (0, 0))],
        out_specs=pl.BlockSpec((BM,), lambda i: (i,)),
        out_shape=jax.ShapeDtypeStruct((B,), jnp.float32),
    )(u_id, u_small, i_id, i_small, uWa, uWb, ub1.reshape(1, -1), uW2,
      ub2.reshape(1, -1), iWa, iWb, ib1.reshape(1, -1), iW2,
      ib2.reshape(1, -1))
    return out


# speedup vs baseline: 1.0990x; 1.0990x over previous
import functools

import jax
import jax.numpy as jnp
from jax import lax
from jax.experimental import pallas as pl
from jax.experimental.pallas import tpu as pltpu
from jax.experimental.pallas import tpu_sc as plsc

B = 16384
NW = 32
BPW = B // NW
CHUNK = 64
NCHUNK = BPW // CHUNK
L_UID = 8192
L_SMALL = 8192


def _pack(x):
    d, l = x.shape
    p = 128 // d
    l1 = l // p
    eye = jnp.eye(d, dtype=jnp.float32)
    xt = lax.dot_general(x, eye, (((0,), (0,)), ((), ())),
                         preferred_element_type=jnp.float32)
    return jnp.concatenate([xt[j * l1:(j + 1) * l1] for j in range(p)], axis=1)


def _tp_body(in_ref, out_ref):
    out_ref[...] = _pack(in_ref[...])


def _transpose_pack(table_t, lanes):
    d, v = table_t.shape
    grid = (v + lanes - 1) // lanes
    rows = lanes * d // 128
    return pl.pallas_call(
        _tp_body,
        grid=(grid,),
        in_specs=[pl.BlockSpec((d, lanes), lambda i: (0, i))],
        out_specs=pl.BlockSpec((rows, 128), lambda i: (i, 0)),
        out_shape=jax.ShapeDtypeStruct((grid * rows, 128), jnp.float32),
    )(table_t)


def _tp_body6(a_ref, b_ref, c_ref, d_ref, e_ref, f_ref, ao, bo, co, do_, eo,
              fo):
    refs = (a_ref, b_ref, c_ref, d_ref, e_ref, f_ref)
    x_all = jnp.concatenate([r[...] for r in refs], axis=0)
    k, l = x_all.shape
    eye = jnp.eye(k, dtype=jnp.float32)
    xt = lax.dot_general(x_all, eye, (((0,), (0,)), ((), ())),
                         preferred_element_type=jnp.float32)
    col = 0
    for r, o in zip(refs, (ao, bo, co, do_, eo, fo)):
        d = r.shape[0]
        p = 128 // d
        l1 = l // p
        o[...] = jnp.concatenate(
            [xt[j * l1:(j + 1) * l1, col:col + d] for j in range(p)], axis=1)
        col += d


def _transpose_pack6(tables_t, lanes):
    v = tables_t[0].shape[1]
    grid = (v + lanes - 1) // lanes
    outs, in_specs, out_specs = [], [], []
    for t in tables_t:
        d = t.shape[0]
        rows = lanes * d // 128
        outs.append(jax.ShapeDtypeStruct((grid * rows, 128), jnp.float32))
        in_specs.append(pl.BlockSpec((d, lanes), lambda i: (0, i)))
        out_specs.append(pl.BlockSpec((rows, 128), lambda i: (i, 0)))
    return pl.pallas_call(
        _tp_body6,
        grid=(grid,),
        in_specs=in_specs,
        out_specs=out_specs,
        out_shape=outs,
    )(*tables_t)


_MESH = plsc.VectorSubcoreMesh(core_axis_name="c", subcore_axis_name="s")


def _extract(gbuf, off, r, width, asm, col0):
    for k in range(width // 16):
        asm[r, pl.ds(col0 + 16 * k, 16)] = gbuf[r, pl.ds(off + 16 * k, 16)]


def _sc_small_body(idx_hbm, off_hbm, seg_t, beh_t, act_t, typ_t, des_t,
                   u_out, i_out, idxv, offv, g0, g1, g2, g3, g4, au, ai, sem):
    wid = lax.axis_index("s") * 2 + lax.axis_index("c")
    base = wid * BPW

    z16 = jnp.zeros((16,), jnp.float32)
    for r in range(CHUNK):
        au[r, pl.ds(48, 16)] = z16
        ai[r, pl.ds(32, 16)] = z16
        ai[r, pl.ds(48, 16)] = z16

    def chunk_body(c, carry):
        pltpu.sync_copy(idx_hbm.at[wid, c], idxv)
        pltpu.sync_copy(off_hbm.at[wid, c], offv)
        copies = [
            pltpu.async_copy(seg_t.at[idxv.at[0]], g0, sem),
            pltpu.async_copy(beh_t.at[idxv.at[1]], g1, sem),
            pltpu.async_copy(act_t.at[idxv.at[2]], g2, sem),
            pltpu.async_copy(typ_t.at[idxv.at[3]], g3, sem),
            pltpu.async_copy(des_t.at[idxv.at[4]], g4, sem),
        ]
        for cp in copies:
            cp.wait()
        for g in range(CHUNK // 16):
            ovs = [offv[f, pl.ds(g * 16, 16)] for f in range(5)]
            for rr in range(16):
                r = g * 16 + rr
                _extract(g0, ovs[0][rr], r, 16, au, 0)
                _extract(g1, ovs[1][rr], r, 16, au, 16)
                _extract(g2, ovs[2][rr], r, 16, au, 32)
                _extract(g3, ovs[3][rr], r, 16, ai, 0)
                _extract(g4, ovs[4][rr], r, 16, ai, 16)
        row0 = pl.multiple_of(base + c * CHUNK, CHUNK)
        pltpu.sync_copy(au, u_out.at[pl.ds(row0, CHUNK)])
        pltpu.sync_copy(ai, i_out.at[pl.ds(row0, CHUNK)])
        return carry

    lax.fori_loop(0, NCHUNK, chunk_body, 0)


_sc_small = functools.partial(
    pl.kernel,
    mesh=_MESH,
    out_type=[
        jax.ShapeDtypeStruct((B, 64), jnp.float32),
        jax.ShapeDtypeStruct((B, 64), jnp.float32),
    ],
    scratch_types=[
        pltpu.VMEM((5, CHUNK), jnp.int32),
        pltpu.VMEM((5, CHUNK), jnp.int32),
        pltpu.VMEM((CHUNK, 128), jnp.float32),
        pltpu.VMEM((CHUNK, 128), jnp.float32),
        pltpu.VMEM((CHUNK, 128), jnp.float32),
        pltpu.VMEM((CHUNK, 128), jnp.float32),
        pltpu.VMEM((CHUNK, 128), jnp.float32),
        pltpu.VMEM((CHUNK, 64), jnp.float32),
        pltpu.VMEM((CHUNK, 64), jnp.float32),
        pltpu.SemaphoreType.DMA,
    ],
)(_sc_small_body)


def _sc_ids_body(idx_hbm, off_hbm, uid_t, iid_t, u_out, i_out,
                 idxv, offv, g0, g1, au, ai, sem):
    wid = lax.axis_index("s") * 2 + lax.axis_index("c")
    base = wid * BPW

    def chunk_body(c, carry):
        pltpu.sync_copy(idx_hbm.at[wid, c], idxv)
        pltpu.sync_copy(off_hbm.at[wid, c], offv)
        cu = pltpu.async_copy(uid_t.at[idxv.at[0]], g0, sem)
        ci = pltpu.async_copy(iid_t.at[idxv.at[1]], g1, sem)
        cu.wait()
        ci.wait()
        for g in range(CHUNK // 16):
            ov0 = offv[0, pl.ds(g * 16, 16)]
            ov1 = offv[1, pl.ds(g * 16, 16)]
            for rr in range(16):
                r = g * 16 + rr
                _extract(g0, ov0[rr], r, 64, au, 0)
                _extract(g1, ov1[rr], r, 64, ai, 0)
        row0 = pl.multiple_of(base + c * CHUNK, CHUNK)
        pltpu.sync_copy(au, u_out.at[pl.ds(row0, CHUNK)])
        pltpu.sync_copy(ai, i_out.at[pl.ds(row0, CHUNK)])
        return carry

    lax.fori_loop(0, NCHUNK, chunk_body, 0)


_sc_ids = functools.partial(
    pl.kernel,
    mesh=_MESH,
    out_type=[
        jax.ShapeDtypeStruct((B, 64), jnp.float32),
        jax.ShapeDtypeStruct((B, 64), jnp.float32),
    ],
    scratch_types=[
        pltpu.VMEM((2, CHUNK), jnp.int32),
        pltpu.VMEM((2, CHUNK), jnp.int32),
        pltpu.VMEM((CHUNK, 128), jnp.float32),
        pltpu.VMEM((CHUNK, 128), jnp.float32),
        pltpu.VMEM((CHUNK, 64), jnp.float32),
        pltpu.VMEM((CHUNK, 64), jnp.float32),
        pltpu.SemaphoreType.DMA,
    ],
)(_sc_ids_body)


BM = 2048


def _tc_mlp_body(uid_r, us_r, iid_r, is_r, uWa_r, uWb_r, ub1_r, uW2_r, ub2_r,
                 iWa_r, iWb_r, ib1_r, iW2_r, ib2_r, o_ref):
    dot = functools.partial(jnp.dot, preferred_element_type=jnp.float32)
    uh = jnp.maximum(
        dot(uid_r[...], uWa_r[...]) + dot(us_r[...], uWb_r[...])
        + ub1_r[...], 0.0)
    ur = dot(uh, uW2_r[...]) + ub2_r[...]
    ih = jnp.maximum(
        dot(iid_r[...], iWa_r[...]) + dot(is_r[...], iWb_r[...])
        + ib1_r[...], 0.0)
    ir = dot(ih, iW2_r[...]) + ib2_r[...]
    o_ref[...] = jnp.sum(ur * ir, axis=1)


def kernel(user_features_batch, item_features_batch, user_id_emb, seg_emb,
           beh_emb, act_emb, uW1, ub1, uW2, ub2, item_id_emb, type_emb,
           descrip_emb, iW1, ib1, iW2, ib2):
    iid_t, seg_t, beh_t, act_t, typ_t, des_t = _transpose_pack6(
        [item_id_emb.T, seg_emb.T, beh_emb.T, act_emb.T, type_emb.T,
         descrip_emb.T], L_SMALL)
    uid_t = _transpose_pack(user_id_emb.T, L_UID)

    def _addr(idx, d, lanes):
        l1 = lanes * d // 128
        b, r = idx // lanes, idx % lanes
        return b * l1 + r % l1, (r // l1) * d

    u0, u1, u2, u3 = (user_features_batch[:, k] for k in range(4))
    i0, i1, i2 = (item_features_batch[:, k] for k in range(3))

    small = [_addr(u1, 16, L_SMALL), _addr(u2, 16, L_SMALL),
             _addr(u3, 16, L_SMALL), _addr(i1, 16, L_SMALL),
             _addr(i2, 16, L_SMALL)]
    ids = [_addr(u0, 64, L_UID), _addr(i0, 64, L_SMALL)]

    def _plan(addrs, n):
        rows = jnp.stack([a[0] for a in addrs])
        offs = jnp.stack([a[1] for a in addrs])
        return (rows.reshape(n, NW, NCHUNK, CHUNK).transpose(1, 2, 0, 3),
                offs.reshape(n, NW, NCHUNK, CHUNK).transpose(1, 2, 0, 3))

    idxS, offS = _plan(small, 5)
    idxI, offI = _plan(ids, 2)

    u_small, i_small = _sc_small(idxS, offS, seg_t, beh_t, act_t, typ_t, des_t)
    u_id, i_id = _sc_ids(idxI, offI, uid_t, iid_t)

    uWa = uW1[0:64]
    uWb = jnp.zeros((64, uW1.shape[1]), jnp.float32).at[0:48].set(uW1[64:112])
    iWa = iW1[0:64]
    iWb = jnp.zeros((64, iW1.shape[1]), jnp.float32).at[0:32].set(iW1[64:96])

    full = lambda a: pl.BlockSpec(a.shape, lambda i: tuple(0 for _ in a.shape))
    feat = pl.BlockSpec((BM, 64), lambda i: (i, 0))
    out = pl.pallas_call(
        _tc_mlp_body,
        grid=(B // BM,),
        in_specs=[feat, feat, feat, feat,
                  full(uWa), full(uWb), pl.BlockSpec((1, 56), lambda i: (0, 0)),
                  full(uW2), pl.BlockSpec((1, 64), lambda i: (0, 0)),
                  full(iWa), full(iWb), pl.BlockSpec((1, 48), lambda i: (0, 0)),
                  full(iW2), pl.BlockSpec((1, 64), lambda i: (0, 0))],
        out_specs=pl.BlockSpec((BM,), lambda i: (i,)),
        out_shape=jax.ShapeDtypeStruct((B,), jnp.float32),
    )(u_id, u_small, i_id, i_small, uWa, uWb, ub1.reshape(1, -1), uW2,
      ub2.reshape(1, -1), iWa, iWb, ib1.reshape(1, -1), iW2,
      ib2.reshape(1, -1))
    return out
